# Initial kernel scaffold; baseline (speedup 1.0000x reference)
import jax
import jax.numpy as jnp
from jax.experimental import pallas as pl


def kernel(edge_emb, edge_index, Wf, bf, Wb, bb, gw, gb):
    raise NotImplementedError("write your pallas kernel here")


# speedup vs baseline: 5.9244x; 5.9244x over previous
import jax
import jax.numpy as jnp
from jax import lax
from jax.experimental import pallas as pl
from jax.experimental.pallas import tpu as pltpu
from jax.experimental.pallas import tpu_sc as plsc

N_NODES = 10000
N_PAD = 10240
N_EDGES = 320000
D = 128
CHUNK = 128
N_CHUNKS = N_EDGES // CHUNK
NC = 2
NS = 16
NW = NC * NS


def _scatter_body(emb_hbm, src_hbm, zsum_hbm, zcnt_hbm, ones_hbm,
                  psum_hbm, pcnt_hbm,
                  acc_sh, cnt_sh, idx_v, rows_v, ones_v):
    c = lax.axis_index("c")
    s = lax.axis_index("s")
    w = c * NS + s
    zrows = N_PAD // NS
    pltpu.sync_copy(zsum_hbm.at[pl.ds(s * zrows, zrows)],
                    acc_sh.at[pl.ds(s * zrows, zrows)])
    pltpu.sync_copy(zcnt_hbm.at[pl.ds(s * zrows, zrows)],
                    cnt_sh.at[pl.ds(s * zrows, zrows)])
    pltpu.sync_copy(ones_hbm, ones_v)
    plsc.subcore_barrier()

    n_iter = (N_CHUNKS + NW - 1) // NW

    def body(i, carry):
        ch = w + i * NW

        @pl.when(ch < N_CHUNKS)
        def _():
            pltpu.sync_copy(src_hbm.at[pl.ds(ch * CHUNK, CHUNK)], idx_v)
            pltpu.sync_copy(emb_hbm.at[pl.ds(ch * CHUNK, CHUNK)], rows_v)
            pltpu.sync_copy(rows_v, acc_sh.at[idx_v], add=True)
            pltpu.sync_copy(ones_v, cnt_sh.at[idx_v], add=True)

        return carry

    lax.fori_loop(0, n_iter, body, 0)
    plsc.subcore_barrier()

    @pl.when(s == 0)
    def _():
        pltpu.sync_copy(acc_sh, psum_hbm.at[pl.ds(c * N_PAD, N_PAD)])
        pltpu.sync_copy(cnt_sh, pcnt_hbm.at[pl.ds(c * N_PAD, N_PAD)])


def _make_scatter():
    mesh = plsc.VectorSubcoreMesh(core_axis_name="c", subcore_axis_name="s")
    return pl.kernel(
        _scatter_body,
        mesh=mesh,
        out_type=[
            jax.ShapeDtypeStruct((NC * N_PAD, D), jnp.float32),
            jax.ShapeDtypeStruct((NC * N_PAD,), jnp.float32),
        ],
        scratch_types=[
            pltpu.VMEM_SHARED((N_PAD, D), jnp.float32),
            pltpu.VMEM_SHARED((N_PAD,), jnp.float32),
            pltpu.VMEM((CHUNK,), jnp.int32),
            pltpu.VMEM((CHUNK, D), jnp.float32),
            pltpu.VMEM((CHUNK,), jnp.float32),
        ],
    )


def _combine_body(p0_ref, p1_ref, c0_ref, c1_ref, tbl_ref, cnt_ref):
    tbl_ref[...] = p0_ref[...] + p1_ref[...]
    cnt_ref[...] = c0_ref[...] + c1_ref[...]


def _combine(psum, pcnt2d):
    nb = 10
    rows = N_PAD // nb
    crows = (N_PAD // D) // nb
    return pl.pallas_call(
        _combine_body,
        grid=(nb,),
        in_specs=[
            pl.BlockSpec((rows, D), lambda i: (i, 0)),
            pl.BlockSpec((rows, D), lambda i: (nb + i, 0)),
            pl.BlockSpec((crows, D), lambda i: (i, 0)),
            pl.BlockSpec((crows, D), lambda i: (nb + i, 0)),
        ],
        out_specs=[
            pl.BlockSpec((rows, D), lambda i: (i, 0)),
            pl.BlockSpec((crows, D), lambda i: (i, 0)),
        ],
        out_shape=[
            jax.ShapeDtypeStruct((N_PAD, D), jnp.float32),
            jax.ShapeDtypeStruct((N_PAD // D, D), jnp.float32),
        ],
    )(psum, psum, pcnt2d, pcnt2d)


def _gather_body(tbl_hbm, cnt1d_hbm, idx_all_hbm,
                 g_hbm, cnt_out_hbm,
                 idx_v, rows_v, cout_v, sem, sem2):
    c = lax.axis_index("c")
    s = lax.axis_index("s")
    n_iter = (N_CHUNKS + NS - 1) // NS

    def body(i, carry):
        ch = s + i * NS

        @pl.when(ch < N_CHUNKS)
        def _():
            base = c * N_EDGES + ch * CHUNK
            pltpu.sync_copy(idx_all_hbm.at[pl.ds(base, CHUNK)], idx_v)
            cp1 = pltpu.async_copy(tbl_hbm.at[idx_v], rows_v, sem)
            cp2 = pltpu.async_copy(cnt1d_hbm.at[idx_v], cout_v, sem2)
            cp1.wait()
            cp2.wait()
            pltpu.sync_copy(rows_v, g_hbm.at[pl.ds(base, CHUNK)])
            pltpu.sync_copy(cout_v, cnt_out_hbm.at[pl.ds(base, CHUNK)])

        return carry

    lax.fori_loop(0, n_iter, body, 0)


def _make_gather():
    mesh = plsc.VectorSubcoreMesh(core_axis_name="c", subcore_axis_name="s")
    return pl.kernel(
        _gather_body,
        mesh=mesh,
        out_type=[
            jax.ShapeDtypeStruct((NC * N_EDGES, D), jnp.float32),
            jax.ShapeDtypeStruct((NC * N_EDGES,), jnp.float32),
        ],
        scratch_types=[
            pltpu.VMEM((CHUNK,), jnp.int32),
            pltpu.VMEM((CHUNK, D), jnp.float32),
            pltpu.VMEM((CHUNK,), jnp.float32),
            pltpu.SemaphoreType.DMA,
            pltpu.SemaphoreType.DMA,
        ],
    )


B_EDGE = 2560
R_SC = B_EDGE // D
R_PAD = 24


def _expand_col(S):
    R = S.shape[0]
    X = jnp.reshape(jnp.broadcast_to(S[:, None, :], (R, D, D)), (R * D, D))
    lane = lax.broadcasted_iota(jnp.int32, (R * D, D), 1)
    want = lax.broadcasted_iota(jnp.int32, (R * D, D), 0) % D
    return jnp.sum(jnp.where(lane == want, X, 0.0), axis=1, keepdims=True)


def _final_body(e_ref, gs_ref, gd_ref, cu_ref, cv_ref, src_ref, dst_ref,
                wf_ref, bf_ref, wb_ref, bb_ref, gwt_ref, gb_ref, out_ref):
    e = e_ref[...]
    cntu = _expand_col(cu_ref[...])[:B_EDGE] - 1.0
    sum_u = gs_ref[...] - e
    agg_u = jnp.where(cntu > 0, sum_u / jnp.maximum(cntu, 1.0), 0.0)
    selfv = _expand_col(
        (src_ref[...] == dst_ref[...]).astype(jnp.float32))[:B_EDGE]
    cntv = _expand_col(cv_ref[...])[:B_EDGE] - selfv
    sum_v = gd_ref[...] - selfv * e
    agg_v = jnp.where(cntv > 0, sum_v / jnp.maximum(cntv, 1.0), 0.0)
    af = jnp.dot(agg_u, wf_ref[...], preferred_element_type=jnp.float32) \
        + bf_ref[...]
    ab = jnp.dot(agg_v, wb_ref[...], preferred_element_type=jnp.float32) \
        + bb_ref[...]
    gate = jax.nn.sigmoid(
        jnp.sum((af + ab) * gwt_ref[...], axis=1, keepdims=True) + gb_ref[...])
    out_ref[...] = gate * af + (1.0 - gate) * ab


def _final(edge_emb, g_all, c2d, src2d, dst2d, Wf, bf2, Wb, bb2, gwt, gb2):
    nb = N_EDGES // B_EDGE
    row = lambda off: pl.BlockSpec((B_EDGE, D), lambda i, _o=off: (_o + i, 0))
    scl = lambda off: pl.BlockSpec((R_PAD, D), lambda i, _o=off: (_o + i, 0))
    pin = lambda shape: pl.BlockSpec(shape, lambda i: (0, 0))
    return pl.pallas_call(
        _final_body,
        grid=(nb,),
        in_specs=[
            row(0),
            row(0), row(nb),
            scl(0), scl(nb),
            scl(0), scl(0),
            pin((D, D)), pin((1, D)), pin((D, D)), pin((1, D)),
            pin((1, D)), pin((1, 1)),
        ],
        out_specs=pl.BlockSpec((B_EDGE, D), lambda i: (i, 0)),
        out_shape=jax.ShapeDtypeStruct((N_EDGES, D), jnp.float32),
    )(edge_emb, g_all, g_all, c2d, c2d, src2d, dst2d,
      Wf, bf2, Wb, bb2, gwt, gb2)


def kernel(edge_emb, edge_index, Wf, bf, Wb, bb, gw, gb):
    idx_all = edge_index.astype(jnp.int32).reshape(NC * N_EDGES)
    src = idx_all[:N_EDGES]
    zsum = jnp.zeros((N_PAD, D), jnp.float32)
    zcnt = jnp.zeros((N_PAD,), jnp.float32)
    ones = jnp.ones((CHUNK,), jnp.float32)

    psum, pcnt = _make_scatter()(edge_emb, src, zsum, zcnt, ones)
    tbl, cnt80 = _combine(psum, pcnt.reshape(NC * N_PAD // D, D))
    g_all, c_all = _make_gather()(tbl, cnt80.reshape(N_PAD), idx_all)

    nb = N_EDGES // B_EDGE

    def lane_pack(x, groups):
        x3 = x.reshape(groups * nb, R_SC, D)
        x3 = jnp.pad(x3, ((0, 0), (0, R_PAD - R_SC), (0, 0)))
        return x3.reshape(groups * nb * R_PAD, D)

    return _final(edge_emb, g_all,
                  lane_pack(c_all, NC),
                  lane_pack(src, 1),
                  lane_pack(idx_all[N_EDGES:], 1),
                  Wf, bf.reshape(1, D), Wb, bb.reshape(1, D),
                  gw.reshape(1, D), gb.reshape(1, 1))
